# Initial kernel scaffold; baseline (speedup 1.0000x reference)
import jax
import jax.numpy as jnp
from jax.experimental import pallas as pl


def kernel(logits, labels):
    raise NotImplementedError("write your pallas kernel here")


# speedup vs baseline: 7.7984x; 7.7984x over previous
import math

import jax
import jax.numpy as jnp
from jax.experimental import pallas as pl
from jax.experimental.pallas import tpu as pltpu

_THRESH = float(-math.log(0.7))
_IGNORE = 255


def _ohem_body(lg_ref, lb_ref, out_ref, keys_ref, nh_ref, hs_ref, *, nsteps,
               n_min, rows_per_step, chunk_rows, nchunks):
    b = pl.program_id(0)
    j = pl.program_id(1)
    nj = pl.num_programs(1)
    step = b * nj + j

    @pl.when(step == 0)
    def _init():
        nh_ref[0] = 0
        hs_ref[0] = 0.0

    x = lg_ref[0]
    lab = lb_ref[0]
    c_dim = x.shape[0]

    m = jnp.max(x, axis=0)
    e = jnp.exp(x - m[None, :, :])
    s = jnp.sum(e, axis=0)
    cls_iota = jax.lax.broadcasted_iota(jnp.int32, x.shape, 0)
    xl = jnp.sum(jnp.where(cls_iota == lab[None, :, :], x, 0.0), axis=0)
    valid = lab != _IGNORE
    loss = jnp.where(valid, (m - xl) + jnp.log(s), 0.0)

    hard = loss > _THRESH
    nh_ref[0] += jnp.sum(hard.astype(jnp.int32))
    hs_ref[0] += jnp.sum(jnp.where(hard, loss, 0.0))

    keys = jax.lax.bitcast_convert_type(loss, jnp.int32)
    keys_ref[pl.ds(step * rows_per_step, rows_per_step), :] = keys

    @pl.when(step == nsteps - 1)
    def _finish():
        def count_ge(c):
            def chunk_body(i, acc):
                kc = keys_ref[pl.ds(i * chunk_rows, chunk_rows), :]
                return acc + jnp.sum((kc >= c).astype(jnp.int32))
            return jax.lax.fori_loop(0, nchunks, chunk_body, jnp.int32(0))

        def bit_body(i, t):
            cand = t | (jnp.int32(1) << (jnp.int32(30) - i))
            return jnp.where(count_ge(cand) >= n_min, cand, t)

        t = jax.lax.fori_loop(0, 31, bit_body, jnp.int32(0))

        def tail_body(i, carry):
            cnt, ssum = carry
            kc = keys_ref[pl.ds(i * chunk_rows, chunk_rows), :]
            vc = jax.lax.bitcast_convert_type(kc, jnp.float32)
            gt = kc > t
            return (cnt + jnp.sum(gt.astype(jnp.int32)),
                    ssum + jnp.sum(jnp.where(gt, vc, 0.0)))

        cnt_gt, sum_gt = jax.lax.fori_loop(
            0, nchunks, tail_body, (jnp.int32(0), jnp.float32(0.0)))

        tval = jax.lax.bitcast_convert_type(t, jnp.float32)
        ties = (jnp.int32(n_min) - cnt_gt).astype(jnp.float32)
        topk_mean = (sum_gt + ties * tval) / jnp.float32(n_min)

        n_hard = nh_ref[0]
        hard_mean = hs_ref[0] / jnp.maximum(n_hard, 1).astype(jnp.float32)
        res = jnp.where(n_hard < n_min, topk_mean, hard_mean)
        out_ref[...] = jnp.broadcast_to(res, (1, 1))


def kernel(logits, labels):
    B, C, H, W = logits.shape
    lab = labels.astype(jnp.int32)
    N = B * H * W
    n_min = N // 16

    RS = 16
    nj = H // RS
    nsteps = B * nj
    rows_per_step = (RS * W) // 512
    total_rows = nsteps * rows_per_step
    chunk_rows = min(256, total_rows)
    nchunks = total_rows // chunk_rows

    import functools
    body = functools.partial(
        _ohem_body, nsteps=nsteps, n_min=n_min, rows_per_step=rows_per_step,
        chunk_rows=chunk_rows, nchunks=nchunks)

    out = pl.pallas_call(
        body,
        grid=(B, nj),
        in_specs=[
            pl.BlockSpec((1, C, RS, W), lambda b, j: (b, 0, j, 0)),
            pl.BlockSpec((1, RS, W), lambda b, j: (b, j, 0)),
        ],
        out_specs=pl.BlockSpec((1, 1), lambda b, j: (0, 0)),
        out_shape=jax.ShapeDtypeStruct((1, 1), jnp.float32),
        scratch_shapes=[
            pltpu.VMEM((total_rows, 512), jnp.int32),
            pltpu.SMEM((1,), jnp.int32),
            pltpu.SMEM((1,), jnp.float32),
        ],
        compiler_params=pltpu.CompilerParams(
            dimension_semantics=("arbitrary", "arbitrary"),
        ),
    )(logits, lab)
    return out[0, 0]


# speedup vs baseline: 20.2403x; 2.5954x over previous
import math

import jax
import jax.numpy as jnp
from jax.experimental import pallas as pl
from jax.experimental.pallas import tpu as pltpu

_THRESH = float(-math.log(0.7))
_IGNORE = 255


def _ohem_body(lg_ref, lb_ref, out_ref, keys_ref, nh_ref, hs_ref, *, nsteps,
               n_min, rows_per_step, chunk_rows, nchunks):
    b = pl.program_id(0)
    j = pl.program_id(1)
    nj = pl.num_programs(1)
    step = b * nj + j

    @pl.when(step == 0)
    def _init():
        nh_ref[0] = 0
        hs_ref[0] = 0.0

    x = lg_ref[0]
    lab = lb_ref[0]
    c_dim = x.shape[0]

    m = jnp.max(x, axis=0)
    e = jnp.exp(x - m[None, :, :])
    s = jnp.sum(e, axis=0)
    cls_iota = jax.lax.broadcasted_iota(jnp.int32, x.shape, 0)
    xl = jnp.sum(jnp.where(cls_iota == lab[None, :, :], x, 0.0), axis=0)
    valid = lab != _IGNORE
    loss = jnp.where(valid, (m - xl) + jnp.log(s), 0.0)

    hard = loss > _THRESH
    nh_ref[0] += jnp.sum(hard.astype(jnp.int32))
    hs_ref[0] += jnp.sum(jnp.where(hard, loss, 0.0))

    keys = jax.lax.bitcast_convert_type(loss, jnp.int32)
    keys_ref[pl.ds(step * rows_per_step, rows_per_step), :] = keys

    def _topk_search():
        def count_ge(c):
            def chunk_body(i, acc):
                kc = keys_ref[pl.ds(i * chunk_rows, chunk_rows), :]
                return acc + jnp.sum((kc >= c).astype(jnp.int32))
            return jax.lax.fori_loop(0, nchunks, chunk_body, jnp.int32(0))

        def bit_body(i, t):
            cand = t | (jnp.int32(1) << (jnp.int32(30) - i))
            return jnp.where(count_ge(cand) >= n_min, cand, t)

        t = jax.lax.fori_loop(0, 31, bit_body, jnp.int32(0))

        def tail_body(i, carry):
            cnt, ssum = carry
            kc = keys_ref[pl.ds(i * chunk_rows, chunk_rows), :]
            vc = jax.lax.bitcast_convert_type(kc, jnp.float32)
            gt = kc > t
            return (cnt + jnp.sum(gt.astype(jnp.int32)),
                    ssum + jnp.sum(jnp.where(gt, vc, 0.0)))

        cnt_gt, sum_gt = jax.lax.fori_loop(
            0, nchunks, tail_body, (jnp.int32(0), jnp.float32(0.0)))

        tval = jax.lax.bitcast_convert_type(t, jnp.float32)
        ties = (jnp.int32(n_min) - cnt_gt).astype(jnp.float32)
        topk_mean = (sum_gt + ties * tval) / jnp.float32(n_min)
        out_ref[...] = jnp.broadcast_to(topk_mean, (1, 1))

    @pl.when(step == nsteps - 1)
    def _finish():
        n_hard = nh_ref[0]
        hard_mean = hs_ref[0] / jnp.maximum(n_hard, 1).astype(jnp.float32)

        @pl.when(n_hard >= n_min)
        def _hard_branch():
            out_ref[...] = jnp.broadcast_to(hard_mean, (1, 1))

        @pl.when(n_hard < n_min)
        def _topk_branch():
            _topk_search()


def kernel(logits, labels):
    B, C, H, W = logits.shape
    lab = labels.astype(jnp.int32)
    N = B * H * W
    n_min = N // 16

    RS = 32
    nj = H // RS
    nsteps = B * nj
    rows_per_step = (RS * W) // 512
    total_rows = nsteps * rows_per_step
    chunk_rows = min(256, total_rows)
    nchunks = total_rows // chunk_rows

    import functools
    body = functools.partial(
        _ohem_body, nsteps=nsteps, n_min=n_min, rows_per_step=rows_per_step,
        chunk_rows=chunk_rows, nchunks=nchunks)

    out = pl.pallas_call(
        body,
        grid=(B, nj),
        in_specs=[
            pl.BlockSpec((1, C, RS, W), lambda b, j: (b, 0, j, 0)),
            pl.BlockSpec((1, RS, W), lambda b, j: (b, j, 0)),
        ],
        out_specs=pl.BlockSpec((1, 1), lambda b, j: (0, 0)),
        out_shape=jax.ShapeDtypeStruct((1, 1), jnp.float32),
        scratch_shapes=[
            pltpu.VMEM((total_rows, 512), jnp.int32),
            pltpu.SMEM((1,), jnp.int32),
            pltpu.SMEM((1,), jnp.float32),
        ],
        compiler_params=pltpu.CompilerParams(
            dimension_semantics=("arbitrary", "arbitrary"),
        ),
    )(logits, lab)
    return out[0, 0]


# speedup vs baseline: 22.3903x; 1.1062x over previous
import math

import jax
import jax.numpy as jnp
from jax.experimental import pallas as pl
from jax.experimental.pallas import tpu as pltpu

_THRESH = float(-math.log(0.7))
_IGNORE = 255


def _tree(vals, op):
    while len(vals) > 1:
        nxt = [op(vals[i], vals[i + 1]) for i in range(0, len(vals) - 1, 2)]
        if len(vals) % 2:
            nxt.append(vals[-1])
        vals = nxt
    return vals[0]


def _ohem_body(lg_ref, lb_ref, out_ref, keys_ref, nh_ref, hs_ref, *, nsteps,
               n_min, rows_per_step, chunk_rows, nchunks):
    b = pl.program_id(0)
    j = pl.program_id(1)
    nj = pl.num_programs(1)
    step = b * nj + j

    @pl.when(step == 0)
    def _init():
        nh_ref[...] = jnp.zeros_like(nh_ref)
        hs_ref[...] = jnp.zeros_like(hs_ref)

    c_dim = lg_ref.shape[1]
    rs = lg_ref.shape[2]
    sub = 8

    for r in range(rs // sub):
        xs = [lg_ref[0, c, pl.ds(r * sub, sub), :] for c in range(c_dim)]
        lab = lb_ref[0, pl.ds(r * sub, sub), :]

        m = _tree(list(xs), jnp.maximum)
        es = [jnp.exp(xc - m) for xc in xs]
        s = _tree(es, lambda a, bb: a + bb)
        vals = list(xs)
        level = 0
        while len(vals) > 1:
            bmask = ((lab >> level) & 1) == 1
            nxt = [jnp.where(bmask, vals[i + 1], vals[i])
                   for i in range(0, len(vals) - 1, 2)]
            if len(vals) % 2:
                nxt.append(vals[-1])
            vals = nxt
            level += 1
        xl = vals[0]

        valid = lab != _IGNORE
        loss = jnp.where(valid, (m - xl) + jnp.log(s), 0.0)

        hard = loss > _THRESH
        nh_ref[pl.ds(0, sub), :] += hard.astype(jnp.int32)
        hs_ref[pl.ds(0, sub), :] += jnp.where(hard, loss, 0.0)

        keys = jax.lax.bitcast_convert_type(loss, jnp.int32)
        keys_ref[pl.ds(step * rows_per_step + r * sub, sub), :] = keys

    def _topk_search():
        def count_ge(c):
            def chunk_body(i, acc):
                kc = keys_ref[pl.ds(i * chunk_rows, chunk_rows), :]
                return acc + jnp.sum((kc >= c).astype(jnp.int32))
            return jax.lax.fori_loop(0, nchunks, chunk_body, jnp.int32(0))

        def bit_body(i, t):
            cand = t | (jnp.int32(1) << (jnp.int32(30) - i))
            return jnp.where(count_ge(cand) >= n_min, cand, t)

        t = jax.lax.fori_loop(0, 31, bit_body, jnp.int32(0))

        def tail_body(i, carry):
            cnt, ssum = carry
            kc = keys_ref[pl.ds(i * chunk_rows, chunk_rows), :]
            vc = jax.lax.bitcast_convert_type(kc, jnp.float32)
            gt = kc > t
            return (cnt + jnp.sum(gt.astype(jnp.int32)),
                    ssum + jnp.sum(jnp.where(gt, vc, 0.0)))

        cnt_gt, sum_gt = jax.lax.fori_loop(
            0, nchunks, tail_body, (jnp.int32(0), jnp.float32(0.0)))

        tval = jax.lax.bitcast_convert_type(t, jnp.float32)
        ties = (jnp.int32(n_min) - cnt_gt).astype(jnp.float32)
        topk_mean = (sum_gt + ties * tval) / jnp.float32(n_min)
        out_ref[...] = jnp.broadcast_to(topk_mean, (1, 1))

    @pl.when(step == nsteps - 1)
    def _finish():
        n_hard = jnp.sum(nh_ref[...])
        hard_mean = jnp.sum(hs_ref[...]) / jnp.maximum(n_hard, 1).astype(
            jnp.float32)

        @pl.when(n_hard >= n_min)
        def _hard_branch():
            out_ref[...] = jnp.broadcast_to(hard_mean, (1, 1))

        @pl.when(n_hard < n_min)
        def _topk_branch():
            _topk_search()


def kernel(logits, labels):
    B, C, H, W = logits.shape
    lab = labels.astype(jnp.int32)
    N = B * H * W
    n_min = N // 16

    RS = 32
    nj = H // RS
    nsteps = B * nj
    rows_per_step = (RS * W) // 512
    total_rows = nsteps * rows_per_step
    chunk_rows = min(256, total_rows)
    nchunks = total_rows // chunk_rows

    import functools
    body = functools.partial(
        _ohem_body, nsteps=nsteps, n_min=n_min, rows_per_step=rows_per_step,
        chunk_rows=chunk_rows, nchunks=nchunks)

    out = pl.pallas_call(
        body,
        grid=(B, nj),
        in_specs=[
            pl.BlockSpec((1, C, RS, W), lambda b, j: (b, 0, j, 0)),
            pl.BlockSpec((1, RS, W), lambda b, j: (b, j, 0)),
        ],
        out_specs=pl.BlockSpec((1, 1), lambda b, j: (0, 0)),
        out_shape=jax.ShapeDtypeStruct((1, 1), jnp.float32),
        scratch_shapes=[
            pltpu.VMEM((total_rows, 512), jnp.int32),
            pltpu.VMEM((8, 512), jnp.int32),
            pltpu.VMEM((8, 512), jnp.float32),
        ],
        compiler_params=pltpu.CompilerParams(
            dimension_semantics=("arbitrary", "arbitrary"),
        ),
    )(logits, lab)
    return out[0, 0]


# speedup vs baseline: 30.3041x; 1.3535x over previous
import math

import numpy as np
import jax
import jax.numpy as jnp
from jax.experimental import pallas as pl
from jax.experimental.pallas import tpu as pltpu

_THRESH = float(-math.log(0.7))
_IGNORE = 255


def _tree(vals, op):
    while len(vals) > 1:
        nxt = [op(vals[i], vals[i + 1]) for i in range(0, len(vals) - 1, 2)]
        if len(vals) % 2:
            nxt.append(vals[-1])
        vals = nxt
    return vals[0]


def _ohem_body(lg_ref, lb_ref, out_ref, keys_ref, *, nsteps, n_min,
               rows_per_step, chunk_rows, nchunks, kthresh):
    b = pl.program_id(0)
    j = pl.program_id(1)
    nj = pl.num_programs(1)
    step = b * nj + j

    c_dim = lg_ref.shape[1]
    rs = lg_ref.shape[2]
    sub = 8

    for r in range(rs // sub):
        xs = [lg_ref[0, c, pl.ds(r * sub, sub), :] for c in range(c_dim)]
        lab = lb_ref[0, pl.ds(r * sub, sub), :]

        m = _tree(list(xs), jnp.maximum)
        es = [jnp.exp(xc - m) for xc in xs]
        s = _tree(es, lambda a, bb: a + bb)
        vals = list(xs)
        level = 0
        while len(vals) > 1:
            bmask = ((lab >> level) & 1) == 1
            nxt = [jnp.where(bmask, vals[i + 1], vals[i])
                   for i in range(0, len(vals) - 1, 2)]
            if len(vals) % 2:
                nxt.append(vals[-1])
            vals = nxt
            level += 1
        xl = vals[0]

        valid = lab != _IGNORE
        loss = jnp.where(valid, (m - xl) + jnp.log(s), 0.0)

        keys = jax.lax.bitcast_convert_type(loss, jnp.int32)
        keys_ref[pl.ds(step * rows_per_step + r * sub, sub), :] = keys

    def count_sum_gt(t):
        def body(i, carry):
            cnt, ssum = carry
            kc = keys_ref[pl.ds(i * chunk_rows, chunk_rows), :]
            vc = jax.lax.bitcast_convert_type(kc, jnp.float32)
            gt = kc > t
            return (cnt + jnp.sum(gt.astype(jnp.int32)),
                    ssum + jnp.sum(jnp.where(gt, vc, 0.0)))
        return jax.lax.fori_loop(
            0, nchunks, body, (jnp.int32(0), jnp.float32(0.0)))

    def _topk_search():
        def count_ge(c):
            def chunk_body(i, acc):
                kc = keys_ref[pl.ds(i * chunk_rows, chunk_rows), :]
                return acc + jnp.sum((kc >= c).astype(jnp.int32))
            return jax.lax.fori_loop(0, nchunks, chunk_body, jnp.int32(0))

        def bit_body(i, t):
            cand = t | (jnp.int32(1) << (jnp.int32(30) - i))
            return jnp.where(count_ge(cand) >= n_min, cand, t)

        t = jax.lax.fori_loop(0, 31, bit_body, jnp.int32(0))
        cnt_gt, sum_gt = count_sum_gt(t)

        tval = jax.lax.bitcast_convert_type(t, jnp.float32)
        ties = (jnp.int32(n_min) - cnt_gt).astype(jnp.float32)
        topk_mean = (sum_gt + ties * tval) / jnp.float32(n_min)
        out_ref[...] = jnp.broadcast_to(topk_mean, (1, 1))

    @pl.when(step == nsteps - 1)
    def _finish():
        n_hard, hard_sum = count_sum_gt(jnp.int32(kthresh))
        hard_mean = hard_sum / jnp.maximum(n_hard, 1).astype(jnp.float32)

        @pl.when(n_hard >= n_min)
        def _hard_branch():
            out_ref[...] = jnp.broadcast_to(hard_mean, (1, 1))

        @pl.when(n_hard < n_min)
        def _topk_branch():
            _topk_search()


def kernel(logits, labels):
    B, C, H, W = logits.shape
    lab = labels.astype(jnp.int32)
    N = B * H * W
    n_min = N // 16

    RS = 64
    nj = H // RS
    nsteps = B * nj
    rows_per_step = (RS * W) // 512
    total_rows = nsteps * rows_per_step
    chunk_rows = min(256, total_rows)
    nchunks = total_rows // chunk_rows
    kthresh = int(np.float32(_THRESH).view(np.int32))

    import functools
    body = functools.partial(
        _ohem_body, nsteps=nsteps, n_min=n_min, rows_per_step=rows_per_step,
        chunk_rows=chunk_rows, nchunks=nchunks, kthresh=kthresh)

    out = pl.pallas_call(
        body,
        grid=(B, nj),
        in_specs=[
            pl.BlockSpec((1, C, RS, W), lambda b, j: (b, 0, j, 0)),
            pl.BlockSpec((1, RS, W), lambda b, j: (b, j, 0)),
        ],
        out_specs=pl.BlockSpec((1, 1), lambda b, j: (0, 0)),
        out_shape=jax.ShapeDtypeStruct((1, 1), jnp.float32),
        scratch_shapes=[
            pltpu.VMEM((total_rows, 512), jnp.int32),
        ],
        compiler_params=pltpu.CompilerParams(
            dimension_semantics=("arbitrary", "arbitrary"),
        ),
    )(logits, lab)
    return out[0, 0]


# speedup vs baseline: 38.2088x; 1.2608x over previous
import math

import numpy as np
import jax
import jax.numpy as jnp
from jax.experimental import pallas as pl
from jax.experimental.pallas import tpu as pltpu

_THRESH = float(-math.log(0.7))
_IGNORE = 255


def _tree(vals, op):
    while len(vals) > 1:
        nxt = [op(vals[i], vals[i + 1]) for i in range(0, len(vals) - 1, 2)]
        if len(vals) % 2:
            nxt.append(vals[-1])
        vals = nxt
    return vals[0]


def _ohem_body(lg_ref, lb_ref, out_ref, keys_ref, *, nsteps, n_min,
               rows_per_step, chunk_rows, nchunks, kthresh):
    b = pl.program_id(0)
    j = pl.program_id(1)
    nj = pl.num_programs(1)
    step = b * nj + j

    c_dim = lg_ref.shape[1]
    rs = lg_ref.shape[2]
    sub = 8

    for r in range(rs // sub):
        xs = [lg_ref[0, c, pl.ds(r * sub, sub), :] for c in range(c_dim)]
        lab = lb_ref[0, pl.ds(r * sub, sub), :]

        m = _tree(list(xs), jnp.maximum)
        es = [jnp.exp(xc - m) for xc in xs]
        s = _tree(es, lambda a, bb: a + bb)
        vals = list(xs)
        level = 0
        while len(vals) > 1:
            bmask = ((lab >> level) & 1) == 1
            nxt = [jnp.where(bmask, vals[i + 1], vals[i])
                   for i in range(0, len(vals) - 1, 2)]
            if len(vals) % 2:
                nxt.append(vals[-1])
            vals = nxt
            level += 1
        xl = vals[0]

        valid = lab != _IGNORE
        loss = jnp.where(valid, (m - xl) + jnp.log(s), 0.0)

        keys = jax.lax.bitcast_convert_type(loss, jnp.int32)
        keys_ref[pl.ds(step * rows_per_step + r * sub, sub), :] = keys

    def count_sum_gt(t):
        def body(i, carry):
            cnt, ssum = carry
            kc = keys_ref[pl.ds(i * chunk_rows, chunk_rows), :]
            vc = jax.lax.bitcast_convert_type(kc, jnp.float32)
            gt = kc > t
            return (cnt + jnp.sum(gt.astype(jnp.int32)),
                    ssum + jnp.sum(jnp.where(gt, vc, 0.0)))
        return jax.lax.fori_loop(
            0, nchunks, body, (jnp.int32(0), jnp.float32(0.0)))

    def _topk_search():
        def count_ge(c):
            def chunk_body(i, acc):
                kc = keys_ref[pl.ds(i * chunk_rows, chunk_rows), :]
                return acc + jnp.sum((kc >= c).astype(jnp.int32))
            return jax.lax.fori_loop(0, nchunks, chunk_body, jnp.int32(0))

        def bit_body(i, t):
            cand = t | (jnp.int32(1) << (jnp.int32(30) - i))
            return jnp.where(count_ge(cand) >= n_min, cand, t)

        t = jax.lax.fori_loop(0, 31, bit_body, jnp.int32(0))
        cnt_gt, sum_gt = count_sum_gt(t)

        tval = jax.lax.bitcast_convert_type(t, jnp.float32)
        ties = (jnp.int32(n_min) - cnt_gt).astype(jnp.float32)
        topk_mean = (sum_gt + ties * tval) / jnp.float32(n_min)
        out_ref[...] = jnp.broadcast_to(topk_mean, (1, 1))

    @pl.when(step == nsteps - 1)
    def _finish():
        n_hard, hard_sum = count_sum_gt(jnp.int32(kthresh))
        hard_mean = hard_sum / jnp.maximum(n_hard, 1).astype(jnp.float32)

        @pl.when(n_hard >= n_min)
        def _hard_branch():
            out_ref[...] = jnp.broadcast_to(hard_mean, (1, 1))

        @pl.when(n_hard < n_min)
        def _topk_branch():
            _topk_search()


def kernel(logits, labels):
    B, C, H, W = logits.shape
    lab = labels.astype(jnp.int32)
    N = B * H * W
    n_min = N // 16

    RS = 128
    nj = H // RS
    nsteps = B * nj
    rows_per_step = (RS * W) // 512
    total_rows = nsteps * rows_per_step
    chunk_rows = min(256, total_rows)
    nchunks = total_rows // chunk_rows
    kthresh = int(np.float32(_THRESH).view(np.int32))

    import functools
    body = functools.partial(
        _ohem_body, nsteps=nsteps, n_min=n_min, rows_per_step=rows_per_step,
        chunk_rows=chunk_rows, nchunks=nchunks, kthresh=kthresh)

    out = pl.pallas_call(
        body,
        grid=(B, nj),
        in_specs=[
            pl.BlockSpec((1, C, RS, W), lambda b, j: (b, 0, j, 0)),
            pl.BlockSpec((1, RS, W), lambda b, j: (b, j, 0)),
        ],
        out_specs=pl.BlockSpec((1, 1), lambda b, j: (0, 0)),
        out_shape=jax.ShapeDtypeStruct((1, 1), jnp.float32),
        scratch_shapes=[
            pltpu.VMEM((total_rows, 512), jnp.int32),
        ],
        compiler_params=pltpu.CompilerParams(
            dimension_semantics=("arbitrary", "arbitrary"),
        ),
    )(logits, lab)
    return out[0, 0]


# speedup vs baseline: 42.9420x; 1.1239x over previous
import math

import numpy as np
import jax
import jax.numpy as jnp
from jax.experimental import pallas as pl
from jax.experimental.pallas import tpu as pltpu

_THRESH = float(-math.log(0.7))
_IGNORE = 255


def _tree(vals, op):
    while len(vals) > 1:
        nxt = [op(vals[i], vals[i + 1]) for i in range(0, len(vals) - 1, 2)]
        if len(vals) % 2:
            nxt.append(vals[-1])
        vals = nxt
    return vals[0]


def _ohem_body(lg_ref, lb_ref, out_ref, keys_ref, *, nsteps, n_min,
               rows_per_step, chunk_rows, nchunks, kthresh):
    b = pl.program_id(0)
    j = pl.program_id(1)
    nj = pl.num_programs(1)
    step = b * nj + j

    c_dim = lg_ref.shape[1]
    rs = lg_ref.shape[2]
    sub = 8

    for r in range(rs // sub):
        xs = [lg_ref[0, c, pl.ds(r * sub, sub), :] for c in range(c_dim)]
        lab = lb_ref[0, pl.ds(r * sub, sub), :]

        m = _tree(list(xs), jnp.maximum)
        es = [jnp.exp(xc - m) for xc in xs]
        s = _tree(es, lambda a, bb: a + bb)
        vals = list(xs)
        level = 0
        while len(vals) > 1:
            bmask = ((lab >> level) & 1) == 1
            nxt = [jnp.where(bmask, vals[i + 1], vals[i])
                   for i in range(0, len(vals) - 1, 2)]
            if len(vals) % 2:
                nxt.append(vals[-1])
            vals = nxt
            level += 1
        xl = vals[0]

        valid = lab != _IGNORE
        loss = jnp.where(valid, (m - xl) + jnp.log(s), 0.0)

        keys = jax.lax.bitcast_convert_type(loss, jnp.int32)
        keys_ref[pl.ds(step * rows_per_step + r * sub, sub), :] = keys

    def count_sum_gt(t):
        def body(i, carry):
            cnt, ssum = carry
            kc = keys_ref[pl.ds(i * chunk_rows, chunk_rows), :]
            vc = jax.lax.bitcast_convert_type(kc, jnp.float32)
            gt = kc > t
            return (cnt + jnp.sum(gt.astype(jnp.int32)),
                    ssum + jnp.sum(jnp.where(gt, vc, 0.0)))
        return jax.lax.fori_loop(
            0, nchunks, body, (jnp.int32(0), jnp.float32(0.0)))

    def _topk_search():
        def count_ge(c):
            def chunk_body(i, acc):
                kc = keys_ref[pl.ds(i * chunk_rows, chunk_rows), :]
                return acc + jnp.sum((kc >= c).astype(jnp.int32))
            return jax.lax.fori_loop(0, nchunks, chunk_body, jnp.int32(0))

        def bit_body(i, t):
            cand = t | (jnp.int32(1) << (jnp.int32(30) - i))
            return jnp.where(count_ge(cand) >= n_min, cand, t)

        t = jax.lax.fori_loop(0, 31, bit_body, jnp.int32(0))
        cnt_gt, sum_gt = count_sum_gt(t)

        tval = jax.lax.bitcast_convert_type(t, jnp.float32)
        ties = (jnp.int32(n_min) - cnt_gt).astype(jnp.float32)
        topk_mean = (sum_gt + ties * tval) / jnp.float32(n_min)
        out_ref[...] = jnp.broadcast_to(topk_mean, (1, 1))

    @pl.when(step == nsteps - 1)
    def _finish():
        n_hard, hard_sum = count_sum_gt(jnp.int32(kthresh))
        hard_mean = hard_sum / jnp.maximum(n_hard, 1).astype(jnp.float32)

        @pl.when(n_hard >= n_min)
        def _hard_branch():
            out_ref[...] = jnp.broadcast_to(hard_mean, (1, 1))

        @pl.when(n_hard < n_min)
        def _topk_branch():
            _topk_search()


def kernel(logits, labels):
    B, C, H, W = logits.shape
    lab = labels.astype(jnp.int32)
    N = B * H * W
    n_min = N // 16

    RS = 256
    nj = H // RS
    nsteps = B * nj
    rows_per_step = (RS * W) // 512
    total_rows = nsteps * rows_per_step
    chunk_rows = min(256, total_rows)
    nchunks = total_rows // chunk_rows
    kthresh = int(np.float32(_THRESH).view(np.int32))

    import functools
    body = functools.partial(
        _ohem_body, nsteps=nsteps, n_min=n_min, rows_per_step=rows_per_step,
        chunk_rows=chunk_rows, nchunks=nchunks, kthresh=kthresh)

    out = pl.pallas_call(
        body,
        grid=(B, nj),
        in_specs=[
            pl.BlockSpec((1, C, RS, W), lambda b, j: (b, 0, j, 0)),
            pl.BlockSpec((1, RS, W), lambda b, j: (b, j, 0)),
        ],
        out_specs=pl.BlockSpec((1, 1), lambda b, j: (0, 0)),
        out_shape=jax.ShapeDtypeStruct((1, 1), jnp.float32),
        scratch_shapes=[
            pltpu.VMEM((total_rows, 512), jnp.int32),
        ],
        compiler_params=pltpu.CompilerParams(
            dimension_semantics=("arbitrary", "arbitrary"),
        ),
    )(logits, lab)
    return out[0, 0]


# speedup vs baseline: 42.9586x; 1.0004x over previous
import math

import numpy as np
import jax
import jax.numpy as jnp
from jax.experimental import pallas as pl
from jax.experimental.pallas import tpu as pltpu

_THRESH = float(-math.log(0.7))
_IGNORE = 255


def _tree(vals, op):
    while len(vals) > 1:
        nxt = [op(vals[i], vals[i + 1]) for i in range(0, len(vals) - 1, 2)]
        if len(vals) % 2:
            nxt.append(vals[-1])
        vals = nxt
    return vals[0]


def _ohem_body(lg_ref, lb_ref, out_ref, keys_ref, *, nsteps, n_min,
               rows_per_step, chunk_rows, nchunks, kthresh):
    b = pl.program_id(0)
    j = pl.program_id(1)
    nj = pl.num_programs(1)
    step = b * nj + j

    c_dim = lg_ref.shape[1]
    rs = lg_ref.shape[2]
    sub = 8

    for r in range(rs // sub):
        xs = [lg_ref[0, c, pl.ds(r * sub, sub), :] for c in range(c_dim)]
        lab = lb_ref[0, pl.ds(r * sub, sub), :]

        m = _tree(list(xs), jnp.maximum)
        es = [jnp.exp(xc - m) for xc in xs]
        s = _tree(es, lambda a, bb: a + bb)
        vals = list(xs)
        level = 0
        while len(vals) > 1:
            bmask = ((lab >> level) & 1) == 1
            nxt = [jnp.where(bmask, vals[i + 1], vals[i])
                   for i in range(0, len(vals) - 1, 2)]
            if len(vals) % 2:
                nxt.append(vals[-1])
            vals = nxt
            level += 1
        xl = vals[0]

        valid = lab != _IGNORE
        loss = jnp.where(valid, (m - xl) + jnp.log(s), 0.0)

        keys = jax.lax.bitcast_convert_type(loss, jnp.int32)
        keys_ref[pl.ds(step * rows_per_step + r * sub, sub), :] = keys

    def count_sum_gt(t):
        def body(i, carry):
            cnt, ssum = carry
            kc = keys_ref[pl.ds(i * chunk_rows, chunk_rows), :]
            vc = jax.lax.bitcast_convert_type(kc, jnp.float32)
            gt = kc > t
            return (cnt + jnp.sum(gt.astype(jnp.int32)),
                    ssum + jnp.sum(jnp.where(gt, vc, 0.0)))
        return jax.lax.fori_loop(
            0, nchunks, body, (jnp.int32(0), jnp.float32(0.0)))

    def _topk_search():
        def count_ge(c):
            def chunk_body(i, acc):
                kc = keys_ref[pl.ds(i * chunk_rows, chunk_rows), :]
                return acc + jnp.sum((kc >= c).astype(jnp.int32))
            return jax.lax.fori_loop(0, nchunks, chunk_body, jnp.int32(0))

        def bit_body(i, t):
            cand = t | (jnp.int32(1) << (jnp.int32(30) - i))
            return jnp.where(count_ge(cand) >= n_min, cand, t)

        t = jax.lax.fori_loop(0, 31, bit_body, jnp.int32(0))
        cnt_gt, sum_gt = count_sum_gt(t)

        tval = jax.lax.bitcast_convert_type(t, jnp.float32)
        ties = (jnp.int32(n_min) - cnt_gt).astype(jnp.float32)
        topk_mean = (sum_gt + ties * tval) / jnp.float32(n_min)
        out_ref[...] = jnp.broadcast_to(topk_mean, (1, 1))

    @pl.when(step == nsteps - 1)
    def _finish():
        n_hard, hard_sum = count_sum_gt(jnp.int32(kthresh))
        hard_mean = hard_sum / jnp.maximum(n_hard, 1).astype(jnp.float32)

        @pl.when(n_hard >= n_min)
        def _hard_branch():
            out_ref[...] = jnp.broadcast_to(hard_mean, (1, 1))

        @pl.when(n_hard < n_min)
        def _topk_branch():
            _topk_search()


def kernel(logits, labels):
    B, C, H, W = logits.shape
    lab = labels.astype(jnp.int32)
    N = B * H * W
    n_min = N // 16

    RS = min(512, H)
    nj = H // RS
    nsteps = B * nj
    rows_per_step = (RS * W) // 512
    total_rows = nsteps * rows_per_step
    chunk_rows = min(256, total_rows)
    nchunks = total_rows // chunk_rows
    kthresh = int(np.float32(_THRESH).view(np.int32))

    import functools
    body = functools.partial(
        _ohem_body, nsteps=nsteps, n_min=n_min, rows_per_step=rows_per_step,
        chunk_rows=chunk_rows, nchunks=nchunks, kthresh=kthresh)

    out = pl.pallas_call(
        body,
        grid=(B, nj),
        in_specs=[
            pl.BlockSpec((1, C, RS, W), lambda b, j: (b, 0, j, 0)),
            pl.BlockSpec((1, RS, W), lambda b, j: (b, j, 0)),
        ],
        out_specs=pl.BlockSpec((1, 1), lambda b, j: (0, 0)),
        out_shape=jax.ShapeDtypeStruct((1, 1), jnp.float32),
        scratch_shapes=[
            pltpu.VMEM((total_rows, 512), jnp.int32),
        ],
        compiler_params=pltpu.CompilerParams(
            dimension_semantics=("arbitrary", "arbitrary"),
        ),
    )(logits, lab)
    return out[0, 0]
